# baseline (speedup 1.0000x reference)
import jax
import jax.numpy as jnp
from jax.experimental import pallas as pl
from jax.experimental.pallas import tpu as pltpu


def _fcl_body(x_ref, f_ref, o_ref):
    b, cin, hw = x_ref.shape
    kk = f_ref.shape[1] // cin
    k = int(round(kk ** 0.5))
    w = int(round(hw ** 0.5))
    h = hw // w

    lane = jax.lax.broadcasted_iota(jnp.int32, (1, hw), 1)
    oh = lane // w
    ow = lane - oh * w
    masks = {}
    for kh in range(k):
        for kw in range(k):
            dh, dw = kh - 1, kw - 1
            conds = []
            if dh == -1:
                conds.append(oh >= 1)
            elif dh == 1:
                conds.append(oh <= h - 2)
            if dw == -1:
                conds.append(ow >= 1)
            elif dw == 1:
                conds.append(ow <= w - 2)
            if conds:
                valid = conds[0]
                for c in conds[1:]:
                    valid = jnp.logical_and(valid, c)
                masks[(kh, kw)] = valid

    for bi in range(b):
        xb = x_ref[bi].astype(jnp.bfloat16)
        pats = []
        for kh in range(k):
            for kw in range(k):
                off = (kh - 1) * w + (kw - 1)
                if off == 0:
                    xs = xb
                else:
                    xs = jnp.concatenate([xb[:, off:], xb[:, :off]], axis=1)
                if (kh, kw) in masks:
                    xs = jnp.where(masks[(kh, kw)], xs, jnp.zeros_like(xs))
                pats.append(xs)
        patches = jnp.concatenate(pats, axis=0)
        d = jnp.dot(f_ref[...], patches, preferred_element_type=jnp.float32)
        o_ref[bi] = d.astype(jnp.bfloat16)


def kernel(x, weights, filters):
    n, cin, h, w = x.shape
    cout, _, f = weights.shape
    k = filters.shape[-1]
    kk = k * k
    hw = h * w

    fmat = jnp.einsum("oif,fp->opi",
                      weights.astype(jnp.float32),
                      filters.reshape(f, kk).astype(jnp.float32))
    fmat = fmat.reshape(cout, kk * cin).astype(jnp.bfloat16)

    xf = x.reshape(n, cin, hw)

    cost = pl.CostEstimate(
        flops=2 * n * hw * cin * kk * cout,
        transcendentals=0,
        bytes_accessed=xf.size * 4 + fmat.size * 2 + n * cout * hw * 2,
    )

    blk = 32 if n % 32 == 0 else 1
    out = pl.pallas_call(
        _fcl_body,
        out_shape=jax.ShapeDtypeStruct((n, cout, hw), jnp.bfloat16),
        grid=(n // blk,),
        in_specs=[
            pl.BlockSpec((blk, cin, hw), lambda g: (g, 0, 0)),
            pl.BlockSpec((cout, kk * cin), lambda g: (0, 0)),
        ],
        out_specs=pl.BlockSpec((blk, cout, hw), lambda g: (g, 0, 0)),
        compiler_params=pltpu.CompilerParams(
            dimension_semantics=("parallel",),
        ),
        cost_estimate=cost,
    )(xf, fmat)

    return out.astype(jnp.float32).reshape(n, cout, h, w)


# speedup vs baseline: 1.0254x; 1.0254x over previous
import jax
import jax.numpy as jnp
from jax.experimental import pallas as pl
from jax.experimental.pallas import tpu as pltpu


def _fcl_body(x_ref, f_ref, o_ref):
    b, cin, hw = x_ref.shape
    kk = f_ref.shape[1] // cin
    k = int(round(kk ** 0.5))
    w = int(round(hw ** 0.5))
    h = hw // w

    lane = jax.lax.broadcasted_iota(jnp.int32, (1, hw), 1)
    ow = lane % w
    zeros_t = jnp.zeros((), jnp.bfloat16)

    for bi in range(b):
        xb = x_ref[bi].astype(jnp.bfloat16)
        base = {
            -1: jnp.where(ow <= w - 2, xb, zeros_t),
            0: xb,
            1: jnp.where(ow >= 1, xb, zeros_t),
        }
        pats = []
        for kh in range(k):
            for kw in range(k):
                dh, dw = kh - 1, kw - 1
                off = dh * w + dw
                src = base[dw]
                if off == 0:
                    xs = src
                elif off > 0:
                    z = jnp.zeros((cin, off), jnp.bfloat16)
                    xs = jnp.concatenate([src[:, off:], z], axis=1)
                else:
                    z = jnp.zeros((cin, -off), jnp.bfloat16)
                    xs = jnp.concatenate([z, src[:, :off]], axis=1)
                pats.append(xs)
        patches = jnp.concatenate(pats, axis=0)
        d = jnp.dot(f_ref[...], patches, preferred_element_type=jnp.float32)
        o_ref[bi] = d.astype(jnp.bfloat16)


def kernel(x, weights, filters):
    n, cin, h, w = x.shape
    cout, _, f = weights.shape
    k = filters.shape[-1]
    kk = k * k
    hw = h * w

    fmat = jnp.einsum("oif,fp->opi",
                      weights.astype(jnp.float32),
                      filters.reshape(f, kk).astype(jnp.float32))
    fmat = fmat.reshape(cout, kk * cin).astype(jnp.bfloat16)

    xf = x.reshape(n, cin, hw)

    cost = pl.CostEstimate(
        flops=2 * n * hw * cin * kk * cout,
        transcendentals=0,
        bytes_accessed=xf.size * 4 + fmat.size * 2 + n * cout * hw * 2,
    )

    blk = 8 if n % 8 == 0 else 1
    out = pl.pallas_call(
        _fcl_body,
        out_shape=jax.ShapeDtypeStruct((n, cout, hw), jnp.bfloat16),
        grid=(n // blk,),
        in_specs=[
            pl.BlockSpec((blk, cin, hw), lambda g: (g, 0, 0)),
            pl.BlockSpec((cout, kk * cin), lambda g: (0, 0)),
        ],
        out_specs=pl.BlockSpec((blk, cout, hw), lambda g: (g, 0, 0)),
        compiler_params=pltpu.CompilerParams(
            dimension_semantics=("parallel",),
        ),
        cost_estimate=cost,
    )(xf, fmat)

    return out.astype(jnp.float32).reshape(n, cout, h, w)


# speedup vs baseline: 1.0284x; 1.0029x over previous
import jax
import jax.numpy as jnp
from jax.experimental import pallas as pl
from jax.experimental.pallas import tpu as pltpu


def _fcl_body(x_ref, f_ref, o_ref):
    b, cin, hw = x_ref.shape
    kk = f_ref.shape[1] // cin
    k = int(round(kk ** 0.5))
    w = int(round(hw ** 0.5))
    h = hw // w

    lane = jax.lax.broadcasted_iota(jnp.int32, (1, hw), 1)
    ow = lane % w
    zeros_t = jnp.zeros((), jnp.bfloat16)

    for bi in range(b):
        xb = x_ref[bi].astype(jnp.bfloat16)
        base = {
            -1: jnp.where(ow <= w - 2, xb, zeros_t),
            0: xb,
            1: jnp.where(ow >= 1, xb, zeros_t),
        }
        pats = []
        for kh in range(k):
            for kw in range(k):
                dh, dw = kh - 1, kw - 1
                off = dh * w + dw
                src = base[dw]
                if off == 0:
                    xs = src
                elif off > 0:
                    z = jnp.zeros((cin, off), jnp.bfloat16)
                    xs = jnp.concatenate([src[:, off:], z], axis=1)
                else:
                    z = jnp.zeros((cin, -off), jnp.bfloat16)
                    xs = jnp.concatenate([z, src[:, :off]], axis=1)
                pats.append(xs)
        patches = jnp.concatenate(pats, axis=0)
        d = jnp.dot(f_ref[...], patches, preferred_element_type=jnp.float32)
        o_ref[bi] = d.astype(jnp.bfloat16)


def kernel(x, weights, filters):
    n, cin, h, w = x.shape
    cout, _, f = weights.shape
    k = filters.shape[-1]
    kk = k * k
    hw = h * w

    fmat = jnp.einsum("oif,fp->opi",
                      weights.astype(jnp.float32),
                      filters.reshape(f, kk).astype(jnp.float32))
    fmat = fmat.reshape(cout, kk * cin).astype(jnp.bfloat16)

    xf = x.reshape(n, cin, hw)

    cost = pl.CostEstimate(
        flops=2 * n * hw * cin * kk * cout,
        transcendentals=0,
        bytes_accessed=xf.size * 4 + fmat.size * 2 + n * cout * hw * 2,
    )

    blk = 8 if n % 16 == 0 else 1
    inner = n // blk // 2
    out = pl.pallas_call(
        _fcl_body,
        out_shape=jax.ShapeDtypeStruct((n, cout, hw), jnp.bfloat16),
        grid=(2, inner),
        in_specs=[
            pl.BlockSpec((blk, cin, hw), lambda c, g: (c * inner + g, 0, 0)),
            pl.BlockSpec((cout, kk * cin), lambda c, g: (0, 0)),
        ],
        out_specs=pl.BlockSpec((blk, cout, hw),
                               lambda c, g: (c * inner + g, 0, 0)),
        compiler_params=pltpu.CompilerParams(
            dimension_semantics=("parallel", "arbitrary"),
        ),
        cost_estimate=cost,
    )(xf, fmat)

    return out.astype(jnp.float32).reshape(n, cout, h, w)
